# Initial kernel scaffold; baseline (speedup 1.0000x reference)
import jax
import jax.numpy as jnp
from jax.experimental import pallas as pl


def kernel(x, token_table, type_table, ln_gamma, ln_beta):
    raise NotImplementedError("write your pallas kernel here")


# speedup vs baseline: 7.1670x; 7.1670x over previous
import functools
import math

import jax
import jax.numpy as jnp
from jax import lax
from jax.experimental import pallas as pl
from jax.experimental.pallas import tpu as pltpu
from jax.experimental.pallas import tpu_sc as plsc

VOCAB = 100000
D_TOK = 56
D_TYPE = 8
D_MODEL = 64
ROW_BLOCK = 10000
N_BLOCKS = VOCAB // ROW_BLOCK


def _table_body(tok_ref, type_ref, gamma_ref, beta_ref, out_ref):
    i = pl.program_id(0)
    t = (i >= 5).astype(jnp.int32) + (i >= 6).astype(jnp.int32) + (i >= 8).astype(jnp.int32)
    typ = type_ref[...]
    row = jnp.zeros((1, D_TYPE), jnp.float32)
    for k in range(4):
        row = jnp.where(t == k, typ[k : k + 1, :], row)
    combined = jnp.concatenate(
        [tok_ref[...], jnp.broadcast_to(row, (ROW_BLOCK, D_TYPE))], axis=-1
    )
    mean = jnp.mean(combined, axis=-1, keepdims=True)
    var = jnp.mean((combined - mean) ** 2, axis=-1, keepdims=True)
    rstd = lax.rsqrt(var + 1e-5)
    out_ref[...] = ((combined - mean) * rstd * gamma_ref[...] + beta_ref[...]) * math.sqrt(
        float(D_MODEL)
    )


def _build_table(token_table, type_table, ln_gamma, ln_beta):
    return pl.pallas_call(
        _table_body,
        grid=(N_BLOCKS,),
        in_specs=[
            pl.BlockSpec((ROW_BLOCK, D_TOK), lambda i: (i, 0)),
            pl.BlockSpec((4, D_TYPE), lambda i: (0, 0)),
            pl.BlockSpec((1, D_MODEL), lambda i: (0, 0)),
            pl.BlockSpec((1, D_MODEL), lambda i: (0, 0)),
        ],
        out_specs=pl.BlockSpec((ROW_BLOCK, D_MODEL), lambda i: (i, 0)),
        out_shape=jax.ShapeDtypeStruct((VOCAB, D_MODEL), jnp.float32),
    )(token_table, type_table, ln_gamma.reshape(1, D_MODEL), ln_beta.reshape(1, D_MODEL))


_NC = 2
_NS = 16
_NW = _NC * _NS
_IDXW = 128
_G = 4
_CHUNK = _G * _IDXW


def _sc_gather(table, idx2d, n_tok):
    rows_per_w = (n_tok // _IDXW) // _NW
    n_chunks = rows_per_w // _G

    mesh = plsc.VectorSubcoreMesh(core_axis_name="c", subcore_axis_name="s")

    @functools.partial(
        pl.kernel,
        mesh=mesh,
        out_type=jax.ShapeDtypeStruct((n_tok, D_MODEL), jnp.float32),
        compiler_params=pltpu.CompilerParams(use_tc_tiling_on_sc=False),
        scratch_types=[
            pltpu.VMEM((_G, _IDXW), jnp.int32),
            pltpu.VMEM((_CHUNK, D_MODEL), jnp.float32),
            pltpu.SemaphoreType.DMA,
        ],
    )
    def k(table_hbm, idx_hbm, out_hbm, idx_v, rows_v, sem):
        wid = lax.axis_index("s") * _NC + lax.axis_index("c")
        row0 = wid * rows_per_w

        def body(ch, carry):
            irow = row0 + ch * _G
            pltpu.sync_copy(idx_hbm.at[pl.ds(irow, _G)], idx_v)
            cps = []
            for j in range(_G):
                cps.append(
                    pltpu.async_copy(
                        table_hbm.at[idx_v.at[j]],
                        rows_v.at[pl.ds(j * _IDXW, _IDXW)],
                        sem,
                    )
                )
            for cp in cps:
                cp.wait()
            pltpu.sync_copy(rows_v, out_hbm.at[pl.ds(irow * _IDXW, _CHUNK)])
            return carry

        lax.fori_loop(0, n_chunks, body, 0)

    return k(table, idx2d)


def kernel(x, token_table, type_table, ln_gamma, ln_beta):
    b, s = x.shape
    n_tok = b * s
    table = _build_table(token_table, type_table, ln_gamma, ln_beta)
    idx2d = x.reshape(n_tok // _IDXW, _IDXW).astype(jnp.int32)
    out = _sc_gather(table, idx2d, n_tok)
    return out.reshape(b, s, D_MODEL)
